# baseline (speedup 1.0000x reference)
import functools

import jax
import jax.numpy as jnp
from jax import lax
from jax.experimental import pallas as pl
from jax.experimental.pallas import tpu as pltpu
from jax.experimental.pallas import tpu_sc as plsc

N = 100000
D = 128
LANES = 16
CHUNK = 128
NFULL = N // CHUNK
TAIL = N - NFULL * CHUNK
TAIL_BASE = NFULL * CHUNK

N_LEVELS, N_TYPES, N_FEATS, N_EXCH, N_PAIRS = 50, 4, 20, 16, 1000
TFE = N_TYPES * N_FEATS * N_EXCH
FUSED = N_LEVELS * TFE

G = 5
LVL_BLK = N_LEVELS // G
PAIRG = N_PAIRS // G


def _build_body(lt, tt, ft, et, pt, w, b, fused, ppair):
    g = pl.program_id(0)
    dn = (((1,), (1,)), ((), ()))
    wv = w[...]
    ltv = lt[pl.ds(LVL_BLK * g, LVL_BLK), :]
    p_rows = lax.dot_general(ltv, wv[:, 0:25], dn,
                             preferred_element_type=jnp.float32) + b[...]
    p_t = lax.dot_general(tt[...], wv[:, 25:50], dn,
                          preferred_element_type=jnp.float32)
    p_f = lax.dot_general(ft[...], wv[:, 50:75], dn,
                          preferred_element_type=jnp.float32)
    p_e = lax.dot_general(et[...], wv[:, 75:100], dn,
                          preferred_element_type=jnp.float32)
    tfe = (p_t[:, None, None, :] + p_f[None, :, None, :]
           + p_e[None, None, :, :]).reshape(TFE, D)
    fused[...] = tfe[None, :, :] + p_rows[:, None, :]
    ptv = pt[pl.ds(PAIRG * g, PAIRG), :]
    ppair[...] = lax.dot_general(ptv, wv[:, 100:128], dn,
                                 preferred_element_type=jnp.float32)[None]


def _build_tables(lt, tt, ft, et, pt, W, b):
    full = pl.BlockSpec(None, lambda g: (0, 0))
    fused, ppair = pl.pallas_call(
        _build_body,
        grid=(G,),
        in_specs=[
            full, full, full, full, full,
            full, full,
        ],
        out_specs=[
            pl.BlockSpec((LVL_BLK, TFE, D), lambda g: (g, 0, 0)),
            pl.BlockSpec((1, PAIRG, D), lambda g: (g, 0, 0)),
        ],
        out_shape=[
            jax.ShapeDtypeStruct((N_LEVELS, TFE, D), jnp.float32),
            jax.ShapeDtypeStruct((G, PAIRG, D), jnp.float32),
        ],
    )(lt, tt, ft, et, pt, W, b.reshape(1, D))
    return fused.reshape(FUSED, D), ppair.reshape(N_PAIRS, D)


def _combo(dst, lb, tb, eb, fb, rows):
    for k in range(rows // LANES):
        sl = pl.ds(k * LANES, LANES)
        dst[sl] = ((lb[sl] * N_TYPES + tb[sl]) * N_FEATS
                   + fb[sl]) * N_EXCH + eb[sl]


def _gather_sum_body(fused, ppair, li, ti, fi, ei, pi, out,
                     il0, it0, if0, ie0, ip0, ic0,
                     il1, it1, if1, ie1, ip1, ic1,
                     ba0, bb0, ba1, bb1,
                     tl, tt_, tf, te, tp, tc, tba, tbb,
                     isem, gs0, gs1, os0, os1):
    info = plsc.get_sparse_core_info()
    nc = info.num_cores
    nw = nc * info.num_subcores
    wid = lax.axis_index("s") * nc + lax.axis_index("c")
    n = (NFULL - wid + nw - 1) // nw
    max_j = (NFULL + nw - 1) // nw // 2 + 1

    isets = ((il0, it0, if0, ie0, ip0, ic0), (il1, it1, if1, ie1, ip1, ic1))
    bsets = ((ba0, bb0), (ba1, bb1))
    gsems = (gs0, gs1)
    osems = (os0, os1)

    def start(i, s):
        base = (wid + i * nw) * CHUNK
        lb, tb, fb, eb, pb, cb = isets[s]
        cps = [pltpu.async_copy(src.at[pl.ds(base, CHUNK)], dst, isem)
               for src, dst in ((li, lb), (ti, tb), (fi, fb), (ei, eb),
                                (pi, pb))]
        for cp in cps:
            cp.wait()
        _combo(cb, lb, tb, eb, fb, CHUNK)
        pltpu.async_copy(fused.at[cb], bsets[s][0], gsems[s])
        pltpu.async_copy(ppair.at[pb], bsets[s][1], gsems[s])

    def wait_gathers(s):
        ba, bb = bsets[s]
        pltpu.make_async_copy(fused.at[pl.ds(0, CHUNK)], ba, gsems[s]).wait()
        pltpu.make_async_copy(ppair.at[pl.ds(0, CHUNK)], bb, gsems[s]).wait()

    def drain_out(s):
        pltpu.make_async_copy(bsets[s][0], out.at[pl.ds(0, CHUNK)],
                              osems[s]).wait()

    def compute_store(i, s):
        base = (wid + i * nw) * CHUNK
        ba, bb = bsets[s]

        def row_body(r, _):
            for g in range(D // LANES):
                col = g * LANES
                ba[r, pl.ds(col, LANES)] = (ba[r, pl.ds(col, LANES)]
                                            + bb[r, pl.ds(col, LANES)])
            return 0

        lax.fori_loop(0, CHUNK, row_body, 0)
        pltpu.async_copy(ba, out.at[pl.ds(base, CHUNK)], osems[s])

    @pl.when(n > 0)
    def _():
        start(0, 0)

    def loop_body(j, _):
        for p in range(2):
            i = 2 * j + p

            @pl.when(i < n)
            def _():
                wait_gathers(p)

                @pl.when(i + 1 < n)
                def _():
                    @pl.when(i + 1 >= 2)
                    def _():
                        drain_out(1 - p)
                    start(i + 1, 1 - p)

                compute_store(i, p)
        return 0

    lax.fori_loop(0, max_j, loop_body, 0)

    drain_out(0)
    drain_out(1)

    @pl.when(wid == 0)
    def _():
        cps = [pltpu.async_copy(src.at[pl.ds(TAIL_BASE, TAIL)], dst, isem)
               for src, dst in ((li, tl), (ti, tt_), (fi, tf), (ei, te),
                                (pi, tp))]
        for cp in cps:
            cp.wait()
        _combo(tc, tl, tt_, te, tf, TAIL)
        ga = pltpu.async_copy(fused.at[tc], tba, gs0)
        gb = pltpu.async_copy(ppair.at[tp], tbb, gs0)
        ga.wait()
        gb.wait()

        def row_body(r, _):
            for g in range(D // LANES):
                col = g * LANES
                tba[r, pl.ds(col, LANES)] = (tba[r, pl.ds(col, LANES)]
                                             + tbb[r, pl.ds(col, LANES)])
            return 0

        lax.fori_loop(0, TAIL, row_body, 0)
        pltpu.sync_copy(tba, out.at[pl.ds(TAIL_BASE, TAIL)])


def kernel(level_table, type_table, feature_table, exchange_table, pair_table,
           W, b, level_indices, type_indices, feature_indices,
           exchange_indices, pair_indices):
    fused, ppair = _build_tables(
        level_table, type_table, feature_table, exchange_table, pair_table,
        W, b)

    mesh = plsc.VectorSubcoreMesh(core_axis_name="c", subcore_axis_name="s")
    scratch = (
        [pltpu.VMEM((CHUNK,), jnp.int32) for _ in range(12)]
        + [pltpu.VMEM((CHUNK, D), jnp.float32) for _ in range(4)]
        + [pltpu.VMEM((TAIL,), jnp.int32) for _ in range(6)]
        + [pltpu.VMEM((TAIL, D), jnp.float32) for _ in range(2)]
        + [pltpu.SemaphoreType.DMA] * 5
    )
    gather_sum = functools.partial(
        pl.kernel,
        out_type=jax.ShapeDtypeStruct((N, D), jnp.float32),
        mesh=mesh,
        scratch_types=scratch,
    )(_gather_sum_body)
    return gather_sum(fused, ppair, level_indices, type_indices,
                      feature_indices, exchange_indices, pair_indices)


# speedup vs baseline: 1.0052x; 1.0052x over previous
import functools

import jax
import jax.numpy as jnp
from jax import lax
from jax.experimental import pallas as pl
from jax.experimental.pallas import tpu as pltpu
from jax.experimental.pallas import tpu_sc as plsc

N = 100000
D = 128
LANES = 16
CHUNK = 128
NFULL = N // CHUNK
TAIL = N - NFULL * CHUNK
TAIL_BASE = NFULL * CHUNK

N_LEVELS, N_TYPES, N_FEATS, N_EXCH, N_PAIRS = 50, 4, 20, 16, 1000
TFE = N_TYPES * N_FEATS * N_EXCH
FUSED = N_LEVELS * TFE

G = 5
LVL_BLK = N_LEVELS // G
PAIRG = N_PAIRS // G


def _build_body(lt, tt, ft, et, pt, w, b, fused, ppair):
    dn = (((1,), (1,)), ((), ()))
    wv = w[...]
    p_rows = lax.dot_general(lt[0], wv[:, 0:25], dn,
                             preferred_element_type=jnp.float32) + b[...]
    p_t = lax.dot_general(tt[...], wv[:, 25:50], dn,
                          preferred_element_type=jnp.float32)
    p_f = lax.dot_general(ft[...], wv[:, 50:75], dn,
                          preferred_element_type=jnp.float32)
    p_e = lax.dot_general(et[...], wv[:, 75:100], dn,
                          preferred_element_type=jnp.float32)
    tfe = (p_t[:, None, None, :] + p_f[None, :, None, :]
           + p_e[None, None, :, :]).reshape(TFE, D)
    fused[...] = tfe[None, :, :] + p_rows[:, None, :]
    ppair[...] = lax.dot_general(pt[0], wv[:, 100:128], dn,
                                 preferred_element_type=jnp.float32)[None]


def _build_tables(lt, tt, ft, et, pt, W, b):
    full = pl.BlockSpec(None, lambda g: (0, 0))
    fused, ppair = pl.pallas_call(
        _build_body,
        grid=(G,),
        in_specs=[
            pl.BlockSpec((1, LVL_BLK, 25), lambda g: (g, 0, 0)),
            full, full, full,
            pl.BlockSpec((1, PAIRG, 28), lambda g: (g, 0, 0)),
            full, full,
        ],
        out_specs=[
            pl.BlockSpec((LVL_BLK, TFE, D), lambda g: (g, 0, 0)),
            pl.BlockSpec((1, PAIRG, D), lambda g: (g, 0, 0)),
        ],
        out_shape=[
            jax.ShapeDtypeStruct((N_LEVELS, TFE, D), jnp.float32),
            jax.ShapeDtypeStruct((G, PAIRG, D), jnp.float32),
        ],
    )(lt.reshape(G, LVL_BLK, 25), tt, ft, et, pt.reshape(G, PAIRG, 28),
      W, b.reshape(1, D))
    return fused.reshape(FUSED, D), ppair.reshape(N_PAIRS, D)


def _combo(dst, lb, tb, eb, fb, rows):
    for k in range(rows // LANES):
        sl = pl.ds(k * LANES, LANES)
        dst[sl] = ((lb[sl] * N_TYPES + tb[sl]) * N_FEATS
                   + fb[sl]) * N_EXCH + eb[sl]


def _gather_sum_body(fused, ppair, li, ti, fi, ei, pi, out,
                     il0, it0, if0, ie0, ip0, ic0,
                     il1, it1, if1, ie1, ip1, ic1,
                     ba0, bb0, ba1, bb1,
                     tl, tt_, tf, te, tp, tc, tba, tbb,
                     isem, gs0, gs1, os0, os1):
    info = plsc.get_sparse_core_info()
    nc = info.num_cores
    nw = nc * info.num_subcores
    wid = lax.axis_index("s") * nc + lax.axis_index("c")
    n = (NFULL - wid + nw - 1) // nw
    max_j = (NFULL + nw - 1) // nw // 2 + 1

    isets = ((il0, it0, if0, ie0, ip0, ic0), (il1, it1, if1, ie1, ip1, ic1))
    bsets = ((ba0, bb0), (ba1, bb1))
    gsems = (gs0, gs1)
    osems = (os0, os1)

    def start(i, s):
        base = (wid + i * nw) * CHUNK
        lb, tb, fb, eb, pb, cb = isets[s]
        cps = [pltpu.async_copy(src.at[pl.ds(base, CHUNK)], dst, isem)
               for src, dst in ((li, lb), (ti, tb), (fi, fb), (ei, eb),
                                (pi, pb))]
        for cp in cps:
            cp.wait()
        _combo(cb, lb, tb, eb, fb, CHUNK)
        pltpu.async_copy(fused.at[cb], bsets[s][0], gsems[s])
        pltpu.async_copy(ppair.at[pb], bsets[s][1], gsems[s])

    def wait_gathers(s):
        ba, bb = bsets[s]
        pltpu.make_async_copy(fused.at[pl.ds(0, CHUNK)], ba, gsems[s]).wait()
        pltpu.make_async_copy(ppair.at[pl.ds(0, CHUNK)], bb, gsems[s]).wait()

    def drain_out(s):
        pltpu.make_async_copy(bsets[s][0], out.at[pl.ds(0, CHUNK)],
                              osems[s]).wait()

    def compute_store(i, s):
        base = (wid + i * nw) * CHUNK
        ba, bb = bsets[s]

        def row_body(r, _):
            for g in range(D // LANES):
                col = g * LANES
                ba[r, pl.ds(col, LANES)] = (ba[r, pl.ds(col, LANES)]
                                            + bb[r, pl.ds(col, LANES)])
            return 0

        lax.fori_loop(0, CHUNK, row_body, 0)
        pltpu.async_copy(ba, out.at[pl.ds(base, CHUNK)], osems[s])

    @pl.when(n > 0)
    def _():
        start(0, 0)

    def loop_body(j, _):
        for p in range(2):
            i = 2 * j + p

            @pl.when(i < n)
            def _():
                wait_gathers(p)

                @pl.when(i + 1 < n)
                def _():
                    @pl.when(i + 1 >= 2)
                    def _():
                        drain_out(1 - p)
                    start(i + 1, 1 - p)

                compute_store(i, p)
        return 0

    lax.fori_loop(0, max_j, loop_body, 0)

    drain_out(0)
    drain_out(1)

    @pl.when(wid == 0)
    def _():
        cps = [pltpu.async_copy(src.at[pl.ds(TAIL_BASE, TAIL)], dst, isem)
               for src, dst in ((li, tl), (ti, tt_), (fi, tf), (ei, te),
                                (pi, tp))]
        for cp in cps:
            cp.wait()
        _combo(tc, tl, tt_, te, tf, TAIL)
        ga = pltpu.async_copy(fused.at[tc], tba, gs0)
        gb = pltpu.async_copy(ppair.at[tp], tbb, gs0)
        ga.wait()
        gb.wait()

        def row_body(r, _):
            for g in range(D // LANES):
                col = g * LANES
                tba[r, pl.ds(col, LANES)] = (tba[r, pl.ds(col, LANES)]
                                             + tbb[r, pl.ds(col, LANES)])
            return 0

        lax.fori_loop(0, TAIL, row_body, 0)
        pltpu.sync_copy(tba, out.at[pl.ds(TAIL_BASE, TAIL)])


def kernel(level_table, type_table, feature_table, exchange_table, pair_table,
           W, b, level_indices, type_indices, feature_indices,
           exchange_indices, pair_indices):
    fused, ppair = _build_tables(
        level_table, type_table, feature_table, exchange_table, pair_table,
        W, b)

    mesh = plsc.VectorSubcoreMesh(core_axis_name="c", subcore_axis_name="s")
    scratch = (
        [pltpu.VMEM((CHUNK,), jnp.int32) for _ in range(12)]
        + [pltpu.VMEM((CHUNK, D), jnp.float32) for _ in range(4)]
        + [pltpu.VMEM((TAIL,), jnp.int32) for _ in range(6)]
        + [pltpu.VMEM((TAIL, D), jnp.float32) for _ in range(2)]
        + [pltpu.SemaphoreType.DMA] * 5
    )
    gather_sum = functools.partial(
        pl.kernel,
        out_type=jax.ShapeDtypeStruct((N, D), jnp.float32),
        mesh=mesh,
        scratch_types=scratch,
    )(_gather_sum_body)
    return gather_sum(fused, ppair, level_indices, type_indices,
                      feature_indices, exchange_indices, pair_indices)


# speedup vs baseline: 1.0113x; 1.0061x over previous
import functools

import jax
import jax.numpy as jnp
from jax import lax
from jax.experimental import pallas as pl
from jax.experimental.pallas import tpu as pltpu
from jax.experimental.pallas import tpu_sc as plsc

N = 100000
D = 128
LANES = 16
CHUNK = 128
NFULL = N // CHUNK
TAIL = N - NFULL * CHUNK
TAIL_BASE = NFULL * CHUNK

N_LEVELS, N_TYPES, N_FEATS, N_EXCH, N_PAIRS = 50, 4, 20, 16, 1000
TFE = N_TYPES * N_FEATS * N_EXCH
FUSED = N_LEVELS * TFE

G = 5
LVL_BLK = N_LEVELS // G
PAIRG = N_PAIRS // G


def _build_body(lt, tt, ft, et, pt, w, b, fused, ppair):
    dn = (((1,), (1,)), ((), ()))
    wv = w[...]
    p_rows = lax.dot_general(lt[0], wv[:, 0:25], dn,
                             preferred_element_type=jnp.float32) + b[...]
    p_t = lax.dot_general(tt[...], wv[:, 25:50], dn,
                          preferred_element_type=jnp.float32)
    p_f = lax.dot_general(ft[...], wv[:, 50:75], dn,
                          preferred_element_type=jnp.float32)
    p_e = lax.dot_general(et[...], wv[:, 75:100], dn,
                          preferred_element_type=jnp.float32)
    tfe = (p_t[:, None, None, :] + p_f[None, :, None, :]
           + p_e[None, None, :, :]).reshape(TFE, D)
    fused[...] = tfe[None, :, :] + p_rows[:, None, :]
    ppair[...] = lax.dot_general(pt[0], wv[:, 100:128], dn,
                                 preferred_element_type=jnp.float32)[None]


def _build_tables(lt, tt, ft, et, pt, W, b):
    full = pl.BlockSpec(None, lambda g: (0, 0))
    fused, ppair = pl.pallas_call(
        _build_body,
        grid=(G,),
        in_specs=[
            pl.BlockSpec((1, LVL_BLK, 25), lambda g: (g, 0, 0)),
            full, full, full,
            pl.BlockSpec((1, PAIRG, 28), lambda g: (g, 0, 0)),
            full, full,
        ],
        out_specs=[
            pl.BlockSpec((LVL_BLK, TFE, D), lambda g: (g, 0, 0)),
            pl.BlockSpec((1, PAIRG, D), lambda g: (g, 0, 0)),
        ],
        out_shape=[
            jax.ShapeDtypeStruct((N_LEVELS, TFE, D), jnp.float32),
            jax.ShapeDtypeStruct((G, PAIRG, D), jnp.float32),
        ],
    )(lt.reshape(G, LVL_BLK, 25), tt, ft, et, pt.reshape(G, PAIRG, 28),
      W, b.reshape(1, D))
    return fused.reshape(FUSED, D), ppair.reshape(N_PAIRS, D)


def _combo(dst, lb, tb, eb, fb, rows):
    for k in range(rows // LANES):
        sl = pl.ds(k * LANES, LANES)
        dst[sl] = ((lb[sl] * N_TYPES + tb[sl]) * N_FEATS
                   + fb[sl]) * N_EXCH + eb[sl]


def _gather_sum_body(fused, ppair, li, ti, fi, ei, pi, out,
                     il0, it0, if0, ie0, ip0, ic0,
                     il1, it1, if1, ie1, ip1, ic1,
                     ba0, bb0, ba1, bb1,
                     tl, tt_, tf, te, tp, tc, tba, tbb,
                     isem, gs0, gs1, os0, os1):
    info = plsc.get_sparse_core_info()
    nc = info.num_cores
    nw = nc * info.num_subcores
    wid = lax.axis_index("s") * nc + lax.axis_index("c")
    n = (NFULL - wid + nw - 1) // nw
    max_j = (NFULL + nw - 1) // nw // 2 + 1

    isets = ((il0, it0, if0, ie0, ip0, ic0), (il1, it1, if1, ie1, ip1, ic1))
    bsets = ((ba0, bb0), (ba1, bb1))
    gsems = (gs0, gs1)
    osems = (os0, os1)

    def start(i, s):
        base = (wid + i * nw) * CHUNK
        lb, tb, fb, eb, pb, cb = isets[s]
        cps = [pltpu.async_copy(src.at[pl.ds(base, CHUNK)], dst, isem)
               for src, dst in ((li, lb), (ti, tb), (fi, fb), (ei, eb),
                                (pi, pb))]
        for cp in cps:
            cp.wait()
        _combo(cb, lb, tb, eb, fb, CHUNK)
        pltpu.async_copy(fused.at[cb], bsets[s][0], gsems[s])
        pltpu.async_copy(ppair.at[pb], bsets[s][1], gsems[s])

    def wait_gathers(s):
        ba, bb = bsets[s]
        pltpu.make_async_copy(fused.at[pl.ds(0, CHUNK)], ba, gsems[s]).wait()
        pltpu.make_async_copy(ppair.at[pl.ds(0, CHUNK)], bb, gsems[s]).wait()

    def drain_out(s):
        pltpu.make_async_copy(bsets[s][0], out.at[pl.ds(0, CHUNK)],
                              osems[s]).wait()

    def compute_store(i, s):
        base = (wid + i * nw) * CHUNK
        ba, bb = bsets[s]

        def row_body(r4, _):
            for dr in range(4):
                r = r4 * 4 + dr
                for g in range(D // LANES):
                    col = g * LANES
                    ba[r, pl.ds(col, LANES)] = (ba[r, pl.ds(col, LANES)]
                                                + bb[r, pl.ds(col, LANES)])
            return 0

        lax.fori_loop(0, CHUNK // 4, row_body, 0)
        pltpu.async_copy(ba, out.at[pl.ds(base, CHUNK)], osems[s])

    @pl.when(n > 0)
    def _():
        start(0, 0)

    def loop_body(j, _):
        for p in range(2):
            i = 2 * j + p

            @pl.when(i < n)
            def _():
                wait_gathers(p)

                @pl.when(i + 1 < n)
                def _():
                    @pl.when(i + 1 >= 2)
                    def _():
                        drain_out(1 - p)
                    start(i + 1, 1 - p)

                compute_store(i, p)
        return 0

    lax.fori_loop(0, max_j, loop_body, 0)

    drain_out(0)
    drain_out(1)

    @pl.when(wid == 0)
    def _():
        cps = [pltpu.async_copy(src.at[pl.ds(TAIL_BASE, TAIL)], dst, isem)
               for src, dst in ((li, tl), (ti, tt_), (fi, tf), (ei, te),
                                (pi, tp))]
        for cp in cps:
            cp.wait()
        _combo(tc, tl, tt_, te, tf, TAIL)
        ga = pltpu.async_copy(fused.at[tc], tba, gs0)
        gb = pltpu.async_copy(ppair.at[tp], tbb, gs0)
        ga.wait()
        gb.wait()

        def row_body(r, _):
            for g in range(D // LANES):
                col = g * LANES
                tba[r, pl.ds(col, LANES)] = (tba[r, pl.ds(col, LANES)]
                                             + tbb[r, pl.ds(col, LANES)])
            return 0

        lax.fori_loop(0, TAIL, row_body, 0)
        pltpu.sync_copy(tba, out.at[pl.ds(TAIL_BASE, TAIL)])


def kernel(level_table, type_table, feature_table, exchange_table, pair_table,
           W, b, level_indices, type_indices, feature_indices,
           exchange_indices, pair_indices):
    fused, ppair = _build_tables(
        level_table, type_table, feature_table, exchange_table, pair_table,
        W, b)

    mesh = plsc.VectorSubcoreMesh(core_axis_name="c", subcore_axis_name="s")
    scratch = (
        [pltpu.VMEM((CHUNK,), jnp.int32) for _ in range(12)]
        + [pltpu.VMEM((CHUNK, D), jnp.float32) for _ in range(4)]
        + [pltpu.VMEM((TAIL,), jnp.int32) for _ in range(6)]
        + [pltpu.VMEM((TAIL, D), jnp.float32) for _ in range(2)]
        + [pltpu.SemaphoreType.DMA] * 5
    )
    gather_sum = functools.partial(
        pl.kernel,
        out_type=jax.ShapeDtypeStruct((N, D), jnp.float32),
        mesh=mesh,
        scratch_types=scratch,
    )(_gather_sum_body)
    return gather_sum(fused, ppair, level_indices, type_indices,
                      feature_indices, exchange_indices, pair_indices)
